# baseline (speedup 1.0000x reference)
import functools

import jax
import jax.numpy as jnp
from jax import lax
from jax.experimental import pallas as pl
from jax.experimental.pallas import tpu as pltpu
from jax.experimental.pallas import tpu_sc as plsc

N = 10000
E = 320000
D = 128
NC = 2
NS = 16
NW = NC * NS
CHUNK = 128
NCH = 80
EPW = NCH * CHUNK
EPAD = EPW * NW
NP = 10112
DW = 128

_mesh = plsc.VectorSubcoreMesh(core_axis_name="c", subcore_axis_name="s")


@functools.partial(
    pl.kernel,
    out_type=jax.ShapeDtypeStruct((NC, NP, DW), jnp.float32),
    mesh=_mesh,
    scratch_types=[
        pltpu.VMEM_SHARED((NP, DW), jnp.float32),
        pltpu.VMEM((NCH, CHUNK), jnp.int32),
        pltpu.VMEM((CHUNK, DW), jnp.float32),
    ],
)
def _deg_sc(col_hbm, ones_hbm, zeros_hbm, out_hbm, acc_sp, cidx, ones_v):
    c = lax.axis_index("c")
    s = lax.axis_index("s")
    wid = s * NC + c
    rz = NP // NS
    pltpu.sync_copy(zeros_hbm.at[pl.ds(s * rz, rz)], acc_sp.at[pl.ds(s * rz, rz)])
    pltpu.sync_copy(ones_hbm, ones_v)
    pltpu.sync_copy(col_hbm.at[pl.ds(wid * NCH, NCH)], cidx)
    plsc.subcore_barrier()

    @pl.loop(0, NCH)
    def _(i):
        pltpu.sync_copy(ones_v, acc_sp.at[cidx.at[i]], add=True)

    plsc.subcore_barrier()
    pltpu.sync_copy(acc_sp.at[pl.ds(s * rz, rz)], out_hbm.at[c, pl.ds(s * rz, rz)])


@functools.partial(
    pl.kernel,
    out_type=jax.ShapeDtypeStruct((NC, NP, D), jnp.float32),
    mesh=_mesh,
    scratch_types=[
        pltpu.VMEM_SHARED((NP, D), jnp.float32),
        pltpu.VMEM((NCH // 2, CHUNK), jnp.int32),
        pltpu.VMEM((NCH // 2, CHUNK), jnp.int32),
        pltpu.VMEM((CHUNK, D), jnp.float32),
        pltpu.VMEM((CHUNK, D), jnp.float32),
        pltpu.SemaphoreType.DMA,
        pltpu.SemaphoreType.DMA,
    ],
)
def _agg_sc(y_hbm, row_hbm, col_hbm, zeros_hbm, out_hbm,
            acc_sp, ridx, cidx, rows0, rows1, sem0, sem1):
    c = lax.axis_index("c")
    s = lax.axis_index("s")
    wid = s * NC + c
    rz = NP // NS
    pltpu.sync_copy(zeros_hbm.at[pl.ds(s * rz, rz)], acc_sp.at[pl.ds(s * rz, rz)])
    plsc.subcore_barrier()

    HCH = NCH // 2
    for h in range(2):
        base = wid * NCH + h * HCH
        pltpu.sync_copy(row_hbm.at[pl.ds(base, HCH)], ridx)
        pltpu.sync_copy(col_hbm.at[pl.ds(base, HCH)], cidx)
        pltpu.async_copy(y_hbm.at[ridx.at[0]], rows0, sem0)

        @pl.loop(0, HCH // 2)
        def _(j):
            i0 = 2 * j
            pltpu.async_copy(y_hbm.at[ridx.at[i0 + 1]], rows1, sem1)
            pltpu.make_async_copy(y_hbm.at[ridx.at[i0]], rows0, sem0).wait()
            pltpu.sync_copy(rows0, acc_sp.at[cidx.at[i0]], add=True)

            @pl.when(j < HCH // 2 - 1)
            def _():
                pltpu.async_copy(y_hbm.at[ridx.at[i0 + 2]], rows0, sem0)

            pltpu.make_async_copy(y_hbm.at[ridx.at[i0 + 1]], rows1, sem1).wait()
            pltpu.sync_copy(rows1, acc_sp.at[cidx.at[i0 + 1]], add=True)

    plsc.subcore_barrier()
    pltpu.sync_copy(acc_sp.at[pl.ds(s * rz, rz)], out_hbm.at[c, pl.ds(s * rz, rz)])


_BM = 1000


def _dvec(degp_ref):
    deg = degp_ref[0, :, 0] + degp_ref[1, :, 0] + 2.0
    return lax.rsqrt(deg)[:, None]


def _mm_scale(x, W, degp):
    def body(x_ref, w_ref, degp_ref, o_ref):
        d = _dvec(degp_ref)
        o_ref[...] = d * jnp.dot(x_ref[...], w_ref[...],
                                 preferred_element_type=jnp.float32)

    return pl.pallas_call(
        body,
        grid=(N // _BM,),
        in_specs=[
            pl.BlockSpec((_BM, D), lambda i: (i, 0)),
            pl.BlockSpec((D, D), lambda i: (0, 0)),
            pl.BlockSpec((NC, _BM, DW), lambda i: (0, i, 0)),
        ],
        out_specs=pl.BlockSpec((_BM, D), lambda i: (i, 0)),
        out_shape=jax.ShapeDtypeStruct((N, D), jnp.float32),
    )(x, W, degp)


def _mid(aggp, y1, degp, b1, W2):
    def body(a_ref, y_ref, degp_ref, b_ref, w_ref, o_ref):
        d = _dvec(degp_ref)
        h = d * (a_ref[0] + a_ref[1] + 2.0 * y_ref[...]) + b_ref[...]
        h = jnp.maximum(h, 0.0)
        o_ref[...] = d * jnp.dot(h, w_ref[...],
                                 preferred_element_type=jnp.float32)

    return pl.pallas_call(
        body,
        grid=(N // _BM,),
        in_specs=[
            pl.BlockSpec((NC, _BM, D), lambda i: (0, i, 0)),
            pl.BlockSpec((_BM, D), lambda i: (i, 0)),
            pl.BlockSpec((NC, _BM, DW), lambda i: (0, i, 0)),
            pl.BlockSpec((1, D), lambda i: (0, 0)),
            pl.BlockSpec((D, D), lambda i: (0, 0)),
        ],
        out_specs=pl.BlockSpec((_BM, D), lambda i: (i, 0)),
        out_shape=jax.ShapeDtypeStruct((N, D), jnp.float32),
    )(aggp, y1, degp, b1, W2)


def _post(aggp, y2, degp, b2):
    def body(a_ref, y_ref, degp_ref, b_ref, o_ref):
        d = _dvec(degp_ref)
        o_ref[...] = d * (a_ref[0] + a_ref[1] + 2.0 * y_ref[...]) + b_ref[...]

    return pl.pallas_call(
        body,
        grid=(N // _BM,),
        in_specs=[
            pl.BlockSpec((NC, _BM, D), lambda i: (0, i, 0)),
            pl.BlockSpec((_BM, D), lambda i: (i, 0)),
            pl.BlockSpec((NC, _BM, DW), lambda i: (0, i, 0)),
            pl.BlockSpec((1, D), lambda i: (0, 0)),
        ],
        out_specs=pl.BlockSpec((_BM, D), lambda i: (i, 0)),
        out_shape=jax.ShapeDtypeStruct((N, D), jnp.float32),
    )(aggp, y2, degp, b2)


def kernel(x, edge_index, W1, b1, W2, b2):
    ei = edge_index.astype(jnp.int32)
    row = jnp.concatenate([ei[0], jnp.zeros((EPAD - E,), jnp.int32)])
    col = jnp.concatenate([ei[1], jnp.full((EPAD - E,), N, jnp.int32)])
    row = row.reshape(NW * NCH, CHUNK)
    col = col.reshape(NW * NCH, CHUNK)
    zeros_d = jnp.zeros((NP, D), jnp.float32)
    zeros_w = jnp.zeros((NP, DW), jnp.float32)
    ones_w = jnp.ones((CHUNK, DW), jnp.float32)

    degp = _deg_sc(col, ones_w, zeros_w)
    y1 = _mm_scale(x, W1, degp)
    aggp1 = _agg_sc(y1, row, col, zeros_d)
    y2 = _mid(aggp1, y1, degp, b1.reshape(1, D), W2)
    aggp2 = _agg_sc(y2, row, col, zeros_d)
    return _post(aggp2, y2, degp, b2.reshape(1, D))


# speedup vs baseline: 1.0105x; 1.0105x over previous
import functools

import jax
import jax.numpy as jnp
from jax import lax
from jax.experimental import pallas as pl
from jax.experimental.pallas import tpu as pltpu
from jax.experimental.pallas import tpu_sc as plsc

N = 10000
E = 320000
D = 128
NC = 2
NS = 16
NW = NC * NS
CHUNK = 128
NCH = 80
HCH = NCH // 2
EPAD = NCH * NW * CHUNK
NP = 10112
DW = 16

_mesh = plsc.VectorSubcoreMesh(core_axis_name="c", subcore_axis_name="s")
_sc_params = pltpu.CompilerParams(use_tc_tiling_on_sc=False)


@functools.partial(
    pl.kernel,
    out_type=jax.ShapeDtypeStruct((NC, NP, DW), jnp.float32),
    mesh=_mesh,
    scratch_types=[
        pltpu.VMEM_SHARED((NP, DW), jnp.float32),
        pltpu.VMEM((NCH, CHUNK), jnp.int32),
        pltpu.VMEM((CHUNK, DW), jnp.float32),
    ],
    compiler_params=_sc_params,
)
def _deg_sc(col_hbm, ones_hbm, zeros_hbm, out_hbm, acc_sp, cidx, ones_v):
    c = lax.axis_index("c")
    s = lax.axis_index("s")
    wid = s * NC + c
    rz = NP // NS
    pltpu.sync_copy(zeros_hbm.at[pl.ds(s * rz, rz)], acc_sp.at[pl.ds(s * rz, rz)])
    pltpu.sync_copy(ones_hbm, ones_v)
    pltpu.sync_copy(col_hbm.at[pl.ds(wid * NCH, NCH)], cidx)
    plsc.subcore_barrier()

    @pl.loop(0, NCH)
    def _(i):
        pltpu.sync_copy(ones_v, acc_sp.at[cidx.at[i]], add=True)

    plsc.subcore_barrier()
    pltpu.sync_copy(acc_sp.at[pl.ds(s * rz, rz)], out_hbm.at[c, pl.ds(s * rz, rz)])


@functools.partial(
    pl.kernel,
    out_type=jax.ShapeDtypeStruct((NC, NP, D), jnp.float32),
    mesh=_mesh,
    scratch_types=[
        pltpu.VMEM_SHARED((NP, D), jnp.float32),
        pltpu.VMEM((HCH, CHUNK), jnp.int32),
        pltpu.VMEM((HCH, CHUNK), jnp.int32),
        pltpu.VMEM((CHUNK, D), jnp.float32),
        pltpu.VMEM((CHUNK, D), jnp.float32),
        pltpu.SemaphoreType.DMA,
        pltpu.SemaphoreType.DMA,
    ],
    compiler_params=_sc_params,
)
def _agg_sc(y_hbm, row_hbm, col_hbm, zeros_hbm, out_hbm,
            acc_sp, ridx, cidx, rows0, rows1, sem0, sem1):
    c = lax.axis_index("c")
    s = lax.axis_index("s")
    wid = s * NC + c
    rz = NP // NS
    sl = pl.ds(s * rz, rz)
    pltpu.sync_copy(zeros_hbm.at[sl], acc_sp.at[sl])
    plsc.subcore_barrier()

    for h in range(2):
        base = wid * NCH + h * HCH
        pltpu.sync_copy(row_hbm.at[pl.ds(base, HCH)], ridx)
        pltpu.sync_copy(col_hbm.at[pl.ds(base, HCH)], cidx)
        pltpu.async_copy(y_hbm.at[ridx.at[0]], rows0, sem0)

        @pl.loop(0, HCH // 2)
        def _(j):
            i0 = 2 * j
            pltpu.async_copy(y_hbm.at[ridx.at[i0 + 1]], rows1, sem1)
            pltpu.make_async_copy(y_hbm.at[ridx.at[i0]], rows0, sem0).wait()
            pltpu.sync_copy(rows0, acc_sp.at[cidx.at[i0]], add=True)

            @pl.when(j < HCH // 2 - 1)
            def _():
                pltpu.async_copy(y_hbm.at[ridx.at[i0 + 2]], rows0, sem0)

            pltpu.make_async_copy(y_hbm.at[ridx.at[i0 + 1]], rows1, sem1).wait()
            pltpu.sync_copy(rows1, acc_sp.at[cidx.at[i0 + 1]], add=True)

    plsc.subcore_barrier()
    pltpu.sync_copy(acc_sp.at[sl], out_hbm.at[c, sl])


_BM = 1000


def _dvec(degp_ref):
    deg = degp_ref[0, :, 0] + degp_ref[1, :, 0] + 2.0
    return lax.rsqrt(deg)[:, None]


def _mm_scale(x, W, degp):
    def body(x_ref, w_ref, degp_ref, o_ref):
        d = _dvec(degp_ref)
        o_ref[...] = d * jnp.dot(x_ref[...], w_ref[...],
                                 preferred_element_type=jnp.float32)

    return pl.pallas_call(
        body,
        grid=(N // _BM,),
        in_specs=[
            pl.BlockSpec((_BM, D), lambda i: (i, 0)),
            pl.BlockSpec((D, D), lambda i: (0, 0)),
            pl.BlockSpec((NC, _BM, DW), lambda i: (0, i, 0)),
        ],
        out_specs=pl.BlockSpec((_BM, D), lambda i: (i, 0)),
        out_shape=jax.ShapeDtypeStruct((N, D), jnp.float32),
    )(x, W, degp)


def _mid(aggp, y1, degp, b1, W2):
    def body(a_ref, y_ref, degp_ref, b_ref, w_ref, o_ref):
        d = _dvec(degp_ref)
        h = d * (a_ref[0] + a_ref[1] + 2.0 * y_ref[...]) + b_ref[...]
        h = jnp.maximum(h, 0.0)
        o_ref[...] = d * jnp.dot(h, w_ref[...],
                                 preferred_element_type=jnp.float32)

    return pl.pallas_call(
        body,
        grid=(N // _BM,),
        in_specs=[
            pl.BlockSpec((NC, _BM, D), lambda i: (0, i, 0)),
            pl.BlockSpec((_BM, D), lambda i: (i, 0)),
            pl.BlockSpec((NC, _BM, DW), lambda i: (0, i, 0)),
            pl.BlockSpec((1, D), lambda i: (0, 0)),
            pl.BlockSpec((D, D), lambda i: (0, 0)),
        ],
        out_specs=pl.BlockSpec((_BM, D), lambda i: (i, 0)),
        out_shape=jax.ShapeDtypeStruct((N, D), jnp.float32),
    )(aggp, y1, degp, b1, W2)


def _post(aggp, y2, degp, b2):
    def body(a_ref, y_ref, degp_ref, b_ref, o_ref):
        d = _dvec(degp_ref)
        o_ref[...] = d * (a_ref[0] + a_ref[1] + 2.0 * y_ref[...]) + b_ref[...]

    return pl.pallas_call(
        body,
        grid=(N // _BM,),
        in_specs=[
            pl.BlockSpec((NC, _BM, D), lambda i: (0, i, 0)),
            pl.BlockSpec((_BM, D), lambda i: (i, 0)),
            pl.BlockSpec((NC, _BM, DW), lambda i: (0, i, 0)),
            pl.BlockSpec((1, D), lambda i: (0, 0)),
        ],
        out_specs=pl.BlockSpec((_BM, D), lambda i: (i, 0)),
        out_shape=jax.ShapeDtypeStruct((N, D), jnp.float32),
    )(aggp, y2, degp, b2)


def kernel(x, edge_index, W1, b1, W2, b2):
    ei = edge_index.astype(jnp.int32)
    row = jnp.concatenate([ei[0], jnp.zeros((EPAD - E,), jnp.int32)])
    col = jnp.concatenate([ei[1], jnp.full((EPAD - E,), N, jnp.int32)])
    row = row.reshape(NW * NCH, CHUNK)
    col = col.reshape(NW * NCH, CHUNK)
    zeros_d = jnp.zeros((NP, D), jnp.float32)
    zeros_w = jnp.zeros((NP, DW), jnp.float32)
    ones_w = jnp.ones((CHUNK, DW), jnp.float32)

    degp = _deg_sc(col, ones_w, zeros_w)
    y1 = _mm_scale(x, W1, degp)
    aggp1 = _agg_sc(y1, row, col, zeros_d)
    y2 = _mid(aggp1, y1, degp, b1.reshape(1, D), W2)
    aggp2 = _agg_sc(y2, row, col, zeros_d)
    return _post(aggp2, y2, degp, b2.reshape(1, D))


# speedup vs baseline: 1.0212x; 1.0106x over previous
import functools

import jax
import jax.numpy as jnp
from jax import lax
from jax.experimental import pallas as pl
from jax.experimental.pallas import tpu as pltpu
from jax.experimental.pallas import tpu_sc as plsc

N = 10000
E = 320000
D = 128
NC = 2
NS = 16
NW = NC * NS
CHUNK = 128
NCH = 80
ACH = 64
ANCH = 160
AR = 40
_DEPTH = 4
EPAD = NCH * NW * CHUNK
NP = 10112
DW = 16

_mesh = plsc.VectorSubcoreMesh(core_axis_name="c", subcore_axis_name="s")
_sc_params = pltpu.CompilerParams(use_tc_tiling_on_sc=False)


@functools.partial(
    pl.kernel,
    out_type=jax.ShapeDtypeStruct((NC, NP, DW), jnp.float32),
    mesh=_mesh,
    scratch_types=[
        pltpu.VMEM_SHARED((NP, DW), jnp.float32),
        pltpu.VMEM((NCH, CHUNK), jnp.int32),
        pltpu.VMEM((CHUNK, DW), jnp.float32),
    ],
    compiler_params=_sc_params,
)
def _deg_sc(col_hbm, ones_hbm, zeros_hbm, out_hbm, acc_sp, cidx, ones_v):
    c = lax.axis_index("c")
    s = lax.axis_index("s")
    wid = s * NC + c
    rz = NP // NS
    pltpu.sync_copy(zeros_hbm.at[pl.ds(s * rz, rz)], acc_sp.at[pl.ds(s * rz, rz)])
    pltpu.sync_copy(ones_hbm, ones_v)
    pltpu.sync_copy(col_hbm.at[pl.ds(wid * NCH, NCH)], cidx)
    plsc.subcore_barrier()

    @pl.loop(0, NCH)
    def _(i):
        pltpu.sync_copy(ones_v, acc_sp.at[cidx.at[i]], add=True)

    plsc.subcore_barrier()
    pltpu.sync_copy(acc_sp.at[pl.ds(s * rz, rz)], out_hbm.at[c, pl.ds(s * rz, rz)])


@functools.partial(
    pl.kernel,
    out_type=jax.ShapeDtypeStruct((NC, NP, D), jnp.float32),
    mesh=_mesh,
    scratch_types=[
        pltpu.VMEM_SHARED((NP, D), jnp.float32),
        pltpu.VMEM((AR, ACH), jnp.int32),
        pltpu.VMEM((AR, ACH), jnp.int32),
    ] + [pltpu.VMEM((ACH, D), jnp.float32)] * _DEPTH
      + [pltpu.SemaphoreType.DMA] * _DEPTH,
    compiler_params=_sc_params,
)
def _agg_sc(y_hbm, row_hbm, col_hbm, zeros_hbm, out_hbm,
            acc_sp, ridx, cidx, *bufs_sems):
    rows = bufs_sems[:_DEPTH]
    sems = bufs_sems[_DEPTH:]
    c = lax.axis_index("c")
    s = lax.axis_index("s")
    wid = s * NC + c
    rz = NP // NS
    sl = pl.ds(s * rz, rz)
    pltpu.sync_copy(zeros_hbm.at[sl], acc_sp.at[sl])
    plsc.subcore_barrier()

    for h in range(ANCH // AR):
        base = wid * ANCH + h * AR
        pltpu.sync_copy(row_hbm.at[pl.ds(base, AR)], ridx)
        pltpu.sync_copy(col_hbm.at[pl.ds(base, AR)], cidx)
        for k in range(_DEPTH - 1):
            pltpu.async_copy(y_hbm.at[ridx.at[k]], rows[k], sems[k])

        @pl.loop(0, AR // _DEPTH)
        def _(jj):
            i0 = jj * _DEPTH
            for k in range(_DEPTH):
                i = i0 + k
                kpre = (k + _DEPTH - 1) % _DEPTH
                @pl.when(i + _DEPTH - 1 < AR)
                def _():
                    pltpu.async_copy(y_hbm.at[ridx.at[i + _DEPTH - 1]],
                                     rows[kpre], sems[kpre])
                pltpu.make_async_copy(y_hbm.at[ridx.at[i]], rows[k], sems[k]).wait()
                pltpu.sync_copy(rows[k], acc_sp.at[cidx.at[i]], add=True)

    plsc.subcore_barrier()
    pltpu.sync_copy(acc_sp.at[sl], out_hbm.at[c, sl])


_BM = 1000


def _dvec(degp_ref):
    deg = degp_ref[0, :, 0] + degp_ref[1, :, 0] + 2.0
    return lax.rsqrt(deg)[:, None]


def _mm_scale(x, W, degp):
    def body(x_ref, w_ref, degp_ref, o_ref):
        d = _dvec(degp_ref)
        o_ref[...] = d * jnp.dot(x_ref[...], w_ref[...],
                                 preferred_element_type=jnp.float32)

    return pl.pallas_call(
        body,
        grid=(N // _BM,),
        in_specs=[
            pl.BlockSpec((_BM, D), lambda i: (i, 0)),
            pl.BlockSpec((D, D), lambda i: (0, 0)),
            pl.BlockSpec((NC, _BM, DW), lambda i: (0, i, 0)),
        ],
        out_specs=pl.BlockSpec((_BM, D), lambda i: (i, 0)),
        out_shape=jax.ShapeDtypeStruct((N, D), jnp.float32),
    )(x, W, degp)


def _mid(aggp, y1, degp, b1, W2):
    def body(a_ref, y_ref, degp_ref, b_ref, w_ref, o_ref):
        d = _dvec(degp_ref)
        h = d * (a_ref[0] + a_ref[1] + 2.0 * y_ref[...]) + b_ref[...]
        h = jnp.maximum(h, 0.0)
        o_ref[...] = d * jnp.dot(h, w_ref[...],
                                 preferred_element_type=jnp.float32)

    return pl.pallas_call(
        body,
        grid=(N // _BM,),
        in_specs=[
            pl.BlockSpec((NC, _BM, D), lambda i: (0, i, 0)),
            pl.BlockSpec((_BM, D), lambda i: (i, 0)),
            pl.BlockSpec((NC, _BM, DW), lambda i: (0, i, 0)),
            pl.BlockSpec((1, D), lambda i: (0, 0)),
            pl.BlockSpec((D, D), lambda i: (0, 0)),
        ],
        out_specs=pl.BlockSpec((_BM, D), lambda i: (i, 0)),
        out_shape=jax.ShapeDtypeStruct((N, D), jnp.float32),
    )(aggp, y1, degp, b1, W2)


def _post(aggp, y2, degp, b2):
    def body(a_ref, y_ref, degp_ref, b_ref, o_ref):
        d = _dvec(degp_ref)
        o_ref[...] = d * (a_ref[0] + a_ref[1] + 2.0 * y_ref[...]) + b_ref[...]

    return pl.pallas_call(
        body,
        grid=(N // _BM,),
        in_specs=[
            pl.BlockSpec((NC, _BM, D), lambda i: (0, i, 0)),
            pl.BlockSpec((_BM, D), lambda i: (i, 0)),
            pl.BlockSpec((NC, _BM, DW), lambda i: (0, i, 0)),
            pl.BlockSpec((1, D), lambda i: (0, 0)),
        ],
        out_specs=pl.BlockSpec((_BM, D), lambda i: (i, 0)),
        out_shape=jax.ShapeDtypeStruct((N, D), jnp.float32),
    )(aggp, y2, degp, b2)


def kernel(x, edge_index, W1, b1, W2, b2):
    ei = edge_index.astype(jnp.int32)
    row = jnp.concatenate([ei[0], jnp.zeros((EPAD - E,), jnp.int32)])
    col = jnp.concatenate([ei[1], jnp.full((EPAD - E,), N, jnp.int32)])
    row_a = row.reshape(NW * ANCH, ACH)
    col_a = col.reshape(NW * ANCH, ACH)
    col = col.reshape(NW * NCH, CHUNK)
    zeros_d = jnp.zeros((NP, D), jnp.float32)
    zeros_w = jnp.zeros((NP, DW), jnp.float32)
    ones_w = jnp.ones((CHUNK, DW), jnp.float32)

    degp = _deg_sc(col, ones_w, zeros_w)
    y1 = _mm_scale(x, W1, degp)
    aggp1 = _agg_sc(y1, row_a, col_a, zeros_d)
    y2 = _mid(aggp1, y1, degp, b1.reshape(1, D), W2)
    aggp2 = _agg_sc(y2, row_a, col_a, zeros_d)
    return _post(aggp2, y2, degp, b2.reshape(1, D))


# speedup vs baseline: 1.3913x; 1.3624x over previous
import functools

import jax
import jax.numpy as jnp
from jax import lax
from jax.experimental import pallas as pl
from jax.experimental.pallas import tpu as pltpu
from jax.experimental.pallas import tpu_sc as plsc

N = 10000
E = 320000
D = 128
DH = D // 2
NC = 2
NS = 16
NW = NC * NS
CHUNK = 128
NCH = 160
RCH = 16
NR = NCH // RCH
EPAD = NCH * NS * CHUNK
NP = 10112
DW = 16

_mesh = plsc.VectorSubcoreMesh(core_axis_name="c", subcore_axis_name="s")
_sc_params = pltpu.CompilerParams(use_tc_tiling_on_sc=False)


@functools.partial(
    pl.kernel,
    out_type=jax.ShapeDtypeStruct((NC, NP, DW), jnp.float32),
    mesh=_mesh,
    scratch_types=[
        pltpu.VMEM_SHARED((NP, DW), jnp.float32),
        pltpu.VMEM((NCH // 2, CHUNK), jnp.int32),
        pltpu.VMEM((CHUNK, DW), jnp.float32),
    ],
    compiler_params=_sc_params,
)
def _deg_sc(col_hbm, ones_hbm, zeros_hbm, out_hbm, acc_sp, cidx, ones_v):
    c = lax.axis_index("c")
    s = lax.axis_index("s")
    wid = s * NC + c
    rz = NP // NS
    pltpu.sync_copy(zeros_hbm.at[pl.ds(s * rz, rz)], acc_sp.at[pl.ds(s * rz, rz)])
    pltpu.sync_copy(ones_hbm, ones_v)
    pltpu.sync_copy(col_hbm.at[pl.ds(wid * (NCH // 2), NCH // 2)], cidx)
    plsc.subcore_barrier()

    @pl.loop(0, NCH // 2)
    def _(i):
        pltpu.sync_copy(ones_v, acc_sp.at[cidx.at[i]], add=True)

    plsc.subcore_barrier()
    pltpu.sync_copy(acc_sp.at[pl.ds(s * rz, rz)], out_hbm.at[c, pl.ds(s * rz, rz)])


_DEPTH = 4

@functools.partial(
    pl.kernel,
    out_type=jax.ShapeDtypeStruct((NC, NP, DH), jnp.float32),
    mesh=_mesh,
    scratch_types=[
        pltpu.VMEM_SHARED((NP, DH), jnp.float32),
        pltpu.VMEM((NCH, CHUNK), jnp.int32),
        pltpu.VMEM((NCH, CHUNK), jnp.int32),
    ] + [pltpu.VMEM((CHUNK, DH), jnp.float32)] * _DEPTH
      + [pltpu.SemaphoreType.DMA] * _DEPTH,
    compiler_params=pltpu.CompilerParams(use_tc_tiling_on_sc=False),
)
def _agg_sc(y_hbm, row_hbm, col_hbm, zeros_hbm, out_hbm,
            acc_sp, ridx, cidx, *bufs_sems):
    rows = bufs_sems[:_DEPTH]
    sems = bufs_sems[_DEPTH:]
    c = lax.axis_index("c")
    s = lax.axis_index("s")
    rz = NP // NS
    sl = pl.ds(s * rz, rz)
    pltpu.sync_copy(zeros_hbm.at[sl], acc_sp.at[sl])
    pltpu.sync_copy(row_hbm.at[pl.ds((c * NS + s) * NCH, NCH)], ridx)
    pltpu.sync_copy(col_hbm.at[pl.ds(s * NCH, NCH)], cidx)
    plsc.subcore_barrier()

    for k in range(_DEPTH - 1):
        pltpu.async_copy(y_hbm.at[ridx.at[k]], rows[k], sems[k])

    @pl.loop(0, NCH // _DEPTH)
    def _(jj):
        i0 = jj * _DEPTH
        for k in range(_DEPTH):
            i = i0 + k
            kpre = (k + _DEPTH - 1) % _DEPTH
            @pl.when(i + _DEPTH - 1 < NCH)
            def _():
                pltpu.async_copy(y_hbm.at[ridx.at[i + _DEPTH - 1]],
                                 rows[kpre], sems[kpre])
            pltpu.make_async_copy(y_hbm.at[ridx.at[i]], rows[k], sems[k]).wait()
            pltpu.sync_copy(rows[k], acc_sp.at[cidx.at[i]], add=True)

    plsc.subcore_barrier()
    pltpu.sync_copy(acc_sp.at[sl], out_hbm.at[c, sl])


_BM = 1000


def _dvec(degp_ref):
    deg = degp_ref[0, :, 0] + degp_ref[1, :, 0] + 2.0
    return lax.rsqrt(deg)[:, None]


def _split(y):
    return jnp.stack([y[:, :DH], y[:, DH:]], axis=0)


def _mm_scale(x, W, degp):
    def body(x_ref, w_ref, degp_ref, o_ref):
        d = _dvec(degp_ref)
        y = d * jnp.dot(x_ref[...], w_ref[...],
                        preferred_element_type=jnp.float32)
        o_ref[...] = _split(y)

    return pl.pallas_call(
        body,
        grid=(N // _BM,),
        in_specs=[
            pl.BlockSpec((_BM, D), lambda i: (i, 0)),
            pl.BlockSpec((D, D), lambda i: (0, 0)),
            pl.BlockSpec((NC, _BM, DW), lambda i: (0, i, 0)),
        ],
        out_specs=pl.BlockSpec((NC, _BM, DH), lambda i: (0, i, 0)),
        out_shape=jax.ShapeDtypeStruct((NC, NP, DH), jnp.float32),
    )(x, W, degp)


def _mid(aggp, y1, degp, b1, W2):
    def body(a_ref, y_ref, degp_ref, b_ref, w_ref, o_ref):
        d = _dvec(degp_ref)
        agg = jnp.concatenate([a_ref[0], a_ref[1]], axis=-1)
        y = jnp.concatenate([y_ref[0], y_ref[1]], axis=-1)
        h = d * (agg + 2.0 * y) + b_ref[...]
        h = jnp.maximum(h, 0.0)
        y2 = d * jnp.dot(h, w_ref[...], preferred_element_type=jnp.float32)
        o_ref[...] = _split(y2)

    return pl.pallas_call(
        body,
        grid=(N // _BM,),
        in_specs=[
            pl.BlockSpec((NC, _BM, DH), lambda i: (0, i, 0)),
            pl.BlockSpec((NC, _BM, DH), lambda i: (0, i, 0)),
            pl.BlockSpec((NC, _BM, DW), lambda i: (0, i, 0)),
            pl.BlockSpec((1, D), lambda i: (0, 0)),
            pl.BlockSpec((D, D), lambda i: (0, 0)),
        ],
        out_specs=pl.BlockSpec((NC, _BM, DH), lambda i: (0, i, 0)),
        out_shape=jax.ShapeDtypeStruct((NC, NP, DH), jnp.float32),
    )(aggp, y1, degp, b1, W2)


def _post(aggp, y2, degp, b2):
    def body(a_ref, y_ref, degp_ref, b_ref, o_ref):
        d = _dvec(degp_ref)
        agg = jnp.concatenate([a_ref[0], a_ref[1]], axis=-1)
        y = jnp.concatenate([y_ref[0], y_ref[1]], axis=-1)
        o_ref[...] = d * (agg + 2.0 * y) + b_ref[...]

    return pl.pallas_call(
        body,
        grid=(N // _BM,),
        in_specs=[
            pl.BlockSpec((NC, _BM, DH), lambda i: (0, i, 0)),
            pl.BlockSpec((NC, _BM, DH), lambda i: (0, i, 0)),
            pl.BlockSpec((NC, _BM, DW), lambda i: (0, i, 0)),
            pl.BlockSpec((1, D), lambda i: (0, 0)),
        ],
        out_specs=pl.BlockSpec((_BM, D), lambda i: (i, 0)),
        out_shape=jax.ShapeDtypeStruct((N, D), jnp.float32),
    )(aggp, y2, degp, b2)


def kernel(x, edge_index, W1, b1, W2, b2):
    ei = edge_index.astype(jnp.int32)
    row = jnp.concatenate([ei[0], jnp.zeros((EPAD - E,), jnp.int32)])
    col = jnp.concatenate([ei[1], jnp.full((EPAD - E,), N, jnp.int32)])
    row2 = jnp.concatenate([row, row + NP]).reshape(2 * NCH * NS, CHUNK)
    col = col.reshape(NCH * NS, CHUNK)
    zeros_h = jnp.zeros((NP, DH), jnp.float32)
    zeros_w = jnp.zeros((NP, DW), jnp.float32)
    ones_w = jnp.ones((CHUNK, DW), jnp.float32)

    degp = _deg_sc(col, ones_w, zeros_w)
    y1 = _mm_scale(x, W1, degp)
    aggp1 = _agg_sc(y1.reshape(NC * NP, DH), row2, col, zeros_h)
    y2 = _mid(aggp1, y1, degp, b1.reshape(1, D), W2)
    aggp2 = _agg_sc(y2.reshape(NC * NP, DH), row2, col, zeros_h)
    return _post(aggp2, y2, degp, b2.reshape(1, D))


# speedup vs baseline: 1.6011x; 1.1508x over previous
import functools

import jax
import jax.numpy as jnp
from jax import lax
from jax.experimental import pallas as pl
from jax.experimental.pallas import tpu as pltpu
from jax.experimental.pallas import tpu_sc as plsc

N = 10000
E = 320000
D = 128
DH = D // 2
NC = 2
NS = 16
NW = NC * NS
CHUNK = 128
NCH = 160
RCH = 16
NR = NCH // RCH
EPAD = NCH * NS * CHUNK
NP = 10112
DW = 128

_mesh = plsc.VectorSubcoreMesh(core_axis_name="c", subcore_axis_name="s")


@functools.partial(
    pl.kernel,
    out_type=jax.ShapeDtypeStruct((NC, NP, DW), jnp.float32),
    mesh=_mesh,
    scratch_types=[
        pltpu.VMEM_SHARED((NP, DW), jnp.float32),
        pltpu.VMEM((NCH // 4, CHUNK), jnp.int32),
        pltpu.VMEM((CHUNK, DW), jnp.float32),
    ],
)
def _deg_sc(col_hbm, ones_hbm, zeros_hbm, out_hbm, acc_sp, cidx, ones_v):
    c = lax.axis_index("c")
    s = lax.axis_index("s")
    wid = s * NC + c
    rz = NP // NS
    pltpu.sync_copy(zeros_hbm.at[pl.ds(s * rz, rz)], acc_sp.at[pl.ds(s * rz, rz)])
    pltpu.sync_copy(ones_hbm, ones_v)
    plsc.subcore_barrier()

    for h in range(2):
        pltpu.sync_copy(
            col_hbm.at[pl.ds(wid * (NCH // 2) + h * (NCH // 4), NCH // 4)], cidx)

        @pl.loop(0, NCH // 4)
        def _(i):
            pltpu.sync_copy(ones_v, acc_sp.at[cidx.at[i]], add=True)

    plsc.subcore_barrier()
    pltpu.sync_copy(acc_sp.at[pl.ds(s * rz, rz)], out_hbm.at[c, pl.ds(s * rz, rz)])


_DEPTH = 4

@functools.partial(
    pl.kernel,
    out_type=jax.ShapeDtypeStruct((NC, NP, DH), jnp.float32),
    mesh=_mesh,
    scratch_types=[
        pltpu.VMEM_SHARED((NP, DH), jnp.float32),
        pltpu.VMEM((NCH, CHUNK), jnp.int32),
        pltpu.VMEM((NCH, CHUNK), jnp.int32),
    ] + [pltpu.VMEM((CHUNK, DH), jnp.float32)] * _DEPTH
      + [pltpu.SemaphoreType.DMA] * _DEPTH,
    compiler_params=pltpu.CompilerParams(use_tc_tiling_on_sc=False),
)
def _agg_sc(y_hbm, row_hbm, col_hbm, zeros_hbm, out_hbm,
            acc_sp, ridx, cidx, *bufs_sems):
    rows = bufs_sems[:_DEPTH]
    sems = bufs_sems[_DEPTH:]
    c = lax.axis_index("c")
    s = lax.axis_index("s")
    rz = NP // NS
    sl = pl.ds(s * rz, rz)
    pltpu.sync_copy(zeros_hbm.at[sl], acc_sp.at[sl])
    pltpu.sync_copy(row_hbm.at[pl.ds((c * NS + s) * NCH, NCH)], ridx)
    pltpu.sync_copy(col_hbm.at[pl.ds(s * NCH, NCH)], cidx)
    plsc.subcore_barrier()

    for k in range(_DEPTH - 1):
        pltpu.async_copy(y_hbm.at[ridx.at[k]], rows[k], sems[k])

    @pl.loop(0, NCH // _DEPTH)
    def _(jj):
        i0 = jj * _DEPTH
        for k in range(_DEPTH):
            i = i0 + k
            kpre = (k + _DEPTH - 1) % _DEPTH
            @pl.when(i + _DEPTH - 1 < NCH)
            def _():
                pltpu.async_copy(y_hbm.at[ridx.at[i + _DEPTH - 1]],
                                 rows[kpre], sems[kpre])
            pltpu.make_async_copy(y_hbm.at[ridx.at[i]], rows[k], sems[k]).wait()
            pltpu.sync_copy(rows[k], acc_sp.at[cidx.at[i]], add=True)

    plsc.subcore_barrier()
    pltpu.sync_copy(acc_sp.at[sl], out_hbm.at[c, sl])


_BM = 1000


def _dvec(degp_ref):
    deg = degp_ref[0, :, 0] + degp_ref[1, :, 0] + 2.0
    return lax.rsqrt(deg)[:, None]


def _split(y):
    return jnp.stack([y[:, :DH], y[:, DH:]], axis=0)


def _mm_scale(x, W, degp):
    def body(x_ref, w_ref, degp_ref, o_ref):
        d = _dvec(degp_ref)
        y = d * jnp.dot(x_ref[...], w_ref[...],
                        preferred_element_type=jnp.float32)
        o_ref[...] = _split(y)

    return pl.pallas_call(
        body,
        grid=(N // _BM,),
        in_specs=[
            pl.BlockSpec((_BM, D), lambda i: (i, 0)),
            pl.BlockSpec((D, D), lambda i: (0, 0)),
            pl.BlockSpec((NC, _BM, DW), lambda i: (0, i, 0)),
        ],
        out_specs=pl.BlockSpec((NC, _BM, DH), lambda i: (0, i, 0)),
        out_shape=jax.ShapeDtypeStruct((NC, NP, DH), jnp.float32),
    )(x, W, degp)


def _mid(aggp, y1, degp, b1, W2):
    def body(a_ref, y_ref, degp_ref, b_ref, w_ref, o_ref):
        d = _dvec(degp_ref)
        agg = jnp.concatenate([a_ref[0], a_ref[1]], axis=-1)
        y = jnp.concatenate([y_ref[0], y_ref[1]], axis=-1)
        h = d * (agg + 2.0 * y) + b_ref[...]
        h = jnp.maximum(h, 0.0)
        y2 = d * jnp.dot(h, w_ref[...], preferred_element_type=jnp.float32)
        o_ref[...] = _split(y2)

    return pl.pallas_call(
        body,
        grid=(N // _BM,),
        in_specs=[
            pl.BlockSpec((NC, _BM, DH), lambda i: (0, i, 0)),
            pl.BlockSpec((NC, _BM, DH), lambda i: (0, i, 0)),
            pl.BlockSpec((NC, _BM, DW), lambda i: (0, i, 0)),
            pl.BlockSpec((1, D), lambda i: (0, 0)),
            pl.BlockSpec((D, D), lambda i: (0, 0)),
        ],
        out_specs=pl.BlockSpec((NC, _BM, DH), lambda i: (0, i, 0)),
        out_shape=jax.ShapeDtypeStruct((NC, NP, DH), jnp.float32),
    )(aggp, y1, degp, b1, W2)


def _post(aggp, y2, degp, b2):
    def body(a_ref, y_ref, degp_ref, b_ref, o_ref):
        d = _dvec(degp_ref)
        agg = jnp.concatenate([a_ref[0], a_ref[1]], axis=-1)
        y = jnp.concatenate([y_ref[0], y_ref[1]], axis=-1)
        o_ref[...] = d * (agg + 2.0 * y) + b_ref[...]

    return pl.pallas_call(
        body,
        grid=(N // _BM,),
        in_specs=[
            pl.BlockSpec((NC, _BM, DH), lambda i: (0, i, 0)),
            pl.BlockSpec((NC, _BM, DH), lambda i: (0, i, 0)),
            pl.BlockSpec((NC, _BM, DW), lambda i: (0, i, 0)),
            pl.BlockSpec((1, D), lambda i: (0, 0)),
        ],
        out_specs=pl.BlockSpec((_BM, D), lambda i: (i, 0)),
        out_shape=jax.ShapeDtypeStruct((N, D), jnp.float32),
    )(aggp, y2, degp, b2)


def kernel(x, edge_index, W1, b1, W2, b2):
    ei = edge_index.astype(jnp.int32)
    row = jnp.concatenate([ei[0], jnp.zeros((EPAD - E,), jnp.int32)])
    col = jnp.concatenate([ei[1], jnp.full((EPAD - E,), N, jnp.int32)])
    row2 = jnp.concatenate([row, row + NP]).reshape(2 * NCH * NS, CHUNK)
    col = col.reshape(NCH * NS, CHUNK)
    zeros_h = jnp.zeros((NP, DH), jnp.float32)
    zeros_w = jnp.zeros((NP, DW), jnp.float32)
    ones_w = jnp.ones((CHUNK, DW), jnp.float32)

    degp = _deg_sc(col, ones_w, zeros_w)
    y1 = _mm_scale(x, W1, degp)
    aggp1 = _agg_sc(y1.reshape(NC * NP, DH), row2, col, zeros_h)
    y2 = _mid(aggp1, y1, degp, b1.reshape(1, D), W2)
    aggp2 = _agg_sc(y2.reshape(NC * NP, DH), row2, col, zeros_h)
    return _post(aggp2, y2, degp, b2.reshape(1, D))
